# baseline (speedup 1.0000x reference)
import functools

import jax
import jax.numpy as jnp
from jax import lax
from jax.experimental import pallas as pl
from jax.experimental.pallas import tpu as pltpu
from jax.experimental.pallas import tpu_sc as plsc

_N = 10000
_E = 160000
_EB = 20000
_KIN = 6
_W = 32
_DEPTH = 4

_TILE = 512
_NTM = (_E + _TILE - 1) // _TILE
_NTB = (_EB + _TILE - 1) // _TILE
_NT = _NTM + _NTB
_EPM = _NTM * _TILE
_EP = _NT * _TILE

_NC = 2
_NS = 16
_NWK = _NC * _NS
_PER_W = _EP // _NWK
_CHUNK = _PER_W // 2
_NCHUNK = _PER_W // _CHUNK

_NCNT = 2 * _N + 16
_NAGG = _N + 16

_f32 = jnp.float32


def _sc_gather_body(h_hbm, src_hbm, out_hbm, idx_v, rows_v, sem):
    c = lax.axis_index("c")
    s = lax.axis_index("s")
    wid = s * _NC + c
    for k in range(_NCHUNK):
        base = wid * _PER_W + k * _CHUNK
        pltpu.sync_copy(src_hbm.at[pl.ds(base, _CHUNK)], idx_v)
        pltpu.async_copy(h_hbm.at[idx_v], rows_v, sem).wait()
        pltpu.sync_copy(rows_v, out_hbm.at[pl.ds(base, _CHUNK)])


def _sc_scatter_body(val_hbm, dst_hbm, z_hbm, out_hbm, val_v, dst_v, acc,
                     *, zrows):
    c = lax.axis_index("c")
    s = lax.axis_index("s")
    wid = s * _NC + c
    zb = s * zrows
    pltpu.sync_copy(z_hbm.at[pl.ds(zb, zrows)], acc.at[pl.ds(zb, zrows)])
    plsc.subcore_barrier()
    for k in range(_NCHUNK):
        base = wid * _PER_W + k * _CHUNK
        pltpu.sync_copy(dst_hbm.at[pl.ds(base, _CHUNK)], dst_v)
        pltpu.sync_copy(val_hbm.at[pl.ds(base, _CHUNK)], val_v)
        pltpu.sync_copy(val_v, acc.at[dst_v], add=True)
    plsc.subcore_barrier()
    pltpu.sync_copy(acc.at[pl.ds(zb, zrows)], out_hbm.at[c, pl.ds(zb, zrows)])


def _sc_mesh():
    return plsc.VectorSubcoreMesh(
        core_axis_name="c", subcore_axis_name="s",
        num_cores=_NC, num_subcores=_NS)


def _gather_rows(table, src_all):
    width = table.shape[1]
    call = pl.kernel(
        _sc_gather_body,
        out_type=jax.ShapeDtypeStruct((_EP, width), _f32),
        mesh=_sc_mesh(),
        compiler_params=pltpu.CompilerParams(use_tc_tiling_on_sc=False),
        scratch_types=[
            pltpu.VMEM((_CHUNK,), jnp.int32),
            pltpu.VMEM((_CHUNK, width), _f32),
            pltpu.SemaphoreType.DMA,
        ],
    )
    return call(table, src_all)


def _scatter_rows(vals, dst, zeros_acc, nacc):
    width = vals.shape[1]
    zrows = nacc // _NS
    body = functools.partial(_sc_scatter_body, zrows=zrows)
    call = pl.kernel(
        body,
        out_type=jax.ShapeDtypeStruct((_NC, nacc, width), _f32),
        mesh=_sc_mesh(),
        compiler_params=pltpu.CompilerParams(use_tc_tiling_on_sc=False),
        scratch_types=[
            pltpu.VMEM((_CHUNK, width), _f32),
            pltpu.VMEM((_CHUNK,), jnp.int32),
            pltpu.VMEM_SHARED((nacc, width), _f32),
        ],
    )
    return call(vals, dst, zeros_acc)


def _wbig_body(ea_ref, w1_ref, b1_ref, w2_ref, b2_ref, w3_ref, o_ref):
    t = jnp.dot(ea_ref[...], w1_ref[0], preferred_element_type=_f32, precision=lax.Precision.HIGHEST) + b1_ref[0]
    t = jnp.maximum(t, 0.0)
    t = jnp.dot(t, w2_ref[0], preferred_element_type=_f32, precision=lax.Precision.HIGHEST) + b2_ref[0]
    t = jnp.maximum(t, 0.0)
    o_ref[...] = jnp.dot(t, w3_ref[0], preferred_element_type=_f32, precision=lax.Precision.HIGHEST)


def _sel(i):
    return jnp.where(i < _NTM, 0, 1)


def _edge_weights(ea_all, w1s, b1s, w2s, b2s, w3s):
    return pl.pallas_call(
        _wbig_body,
        grid=(_NT,),
        in_specs=[
            pl.BlockSpec((_TILE, _KIN), lambda i: (i, 0)),
            pl.BlockSpec((1, _KIN, 64), lambda i: (_sel(i), 0, 0)),
            pl.BlockSpec((1, 1, 64), lambda i: (_sel(i), 0, 0)),
            pl.BlockSpec((1, 64, 64), lambda i: (_sel(i), 0, 0)),
            pl.BlockSpec((1, 1, 64), lambda i: (_sel(i), 0, 0)),
            pl.BlockSpec((1, 64, _W * _W), lambda i: (_sel(i), 0, 0)),
        ],
        out_specs=pl.BlockSpec((_TILE, _W * _W), lambda i: (i, 0)),
        out_shape=jax.ShapeDtypeStruct((_EP, _W * _W), _f32),
    )(ea_all, w1s, b1s, w2s, b2s, w3s)


def _msg_body(xg_ref, w_ref, b3_ref, einv_ref, rep_ref, o_ref):
    xg = xg_ref[...]
    rep = rep_ref[...]
    hi = xg.astype(jnp.bfloat16).astype(_f32)
    lo = xg - hi
    xrep = (jnp.dot(hi, rep, preferred_element_type=_f32)
            + jnp.dot(lo, rep, preferred_element_type=_f32))
    w = w_ref[...]
    acc = xrep[:, 0:128] * w[:, 0:128]
    for q in range(1, 8):
        acc = acc + xrep[:, q * 128:(q + 1) * 128] * w[:, q * 128:(q + 1) * 128]
    b3 = b3_ref[0]
    msg = (acc[:, 0:32] + acc[:, 32:64] + acc[:, 64:96] + acc[:, 96:128]
           + jnp.dot(hi, b3, preferred_element_type=_f32)
           + jnp.dot(lo, b3, preferred_element_type=_f32))
    o_ref[...] = msg * einv_ref[:, 0:1]


def _messages(xg, wbig, b3s, einv, rep):
    return pl.pallas_call(
        _msg_body,
        grid=(_NT,),
        in_specs=[
            pl.BlockSpec((_TILE, _W), lambda i: (i, 0)),
            pl.BlockSpec((_TILE, _W * _W), lambda i: (i, 0)),
            pl.BlockSpec((1, _W, _W), lambda i: (_sel(i), 0, 0)),
            pl.BlockSpec((_TILE, 16), lambda i: (i, 0)),
            pl.BlockSpec((_W, _W * _W), lambda i: (0, 0)),
        ],
        out_specs=pl.BlockSpec((_TILE, _W), lambda i: (i, 0)),
        out_shape=jax.ShapeDtypeStruct((_EP, _W), _f32),
    )(xg, wbig, b3s, einv, rep)


def _skip_body(x4_ref, g_ref, b_ref, o_ref):
    o_ref[...] = (jnp.dot(x4_ref[...], g_ref[...], preferred_element_type=_f32, precision=lax.Precision.HIGHEST)
                  + b_ref[...])


def _skip4(x4, g, b128):
    return pl.pallas_call(
        _skip_body,
        out_shape=jax.ShapeDtypeStruct((_N // 4, 128), _f32),
    )(x4, g, b128)


def _combine_body(p0, p1, h4, sk4, r_ref, bs_ref, o_ref):
    y = (p0[...] + p1[...]
         + jnp.dot(h4[...], r_ref[...], preferred_element_type=_f32, precision=lax.Precision.HIGHEST)
         + bs_ref[...])
    o_ref[...] = jnp.maximum(y, 0.0) + sk4[...]


def _combine(p0, p1, h4, sk4, rblk, bs128):
    return pl.pallas_call(
        _combine_body,
        out_shape=jax.ShapeDtypeStruct((_N // 4, 128), _f32),
    )(p0, p1, h4, sk4, rblk, bs128)


def _inv_body(c0, c1, o_ref):
    o_ref[...] = 1.0 / jnp.maximum(c0[...] + c1[...], 1.0)


def _inv_table(cnt):
    rows = _NCNT * 16 // 128
    c0 = cnt[0].reshape(rows, 128)
    c1 = cnt[1].reshape(rows, 128)
    inv = pl.pallas_call(
        _inv_body,
        out_shape=jax.ShapeDtypeStruct((rows, 128), _f32),
    )(c0, c1)
    return inv.reshape(_NCNT, 16)


def _final_body(h4, f_ref, fb_ref, o_ref):
    o_ref[...] = (jnp.dot(h4[...], f_ref[...], preferred_element_type=_f32, precision=lax.Precision.HIGHEST)
                  + fb_ref[...])


def _final_proj(h4, fblk, fb):
    return pl.pallas_call(
        _final_body,
        out_shape=jax.ShapeDtypeStruct((_N // 4, 4), _f32),
    )(h4, fblk, fb)


def _blockdiag4(m):
    a, b = m.shape
    z = jnp.zeros((4, a, 4, b), m.dtype)
    for j in range(4):
        z = z.at[j, :, j, :].set(m)
    return z.reshape(4 * a, 4 * b)


def kernel(x, edge_index, edge_attr, edge_index_boundary, edge_attr_boundary,
           params):
    src_all = jnp.zeros((_EP,), jnp.int32)
    src_all = src_all.at[:_E].set(edge_index[0])
    src_all = src_all.at[_EPM:_EPM + _EB].set(edge_index_boundary[0])

    dst_cnt = jnp.full((_EP,), 2 * _N, jnp.int32)
    dst_cnt = dst_cnt.at[:_E].set(edge_index[1])
    dst_cnt = dst_cnt.at[_EPM:_EPM + _EB].set(edge_index_boundary[1] + _N)
    dst_agg = jnp.full((_EP,), _N, jnp.int32)
    dst_agg = dst_agg.at[:_E].set(edge_index[1])
    dst_agg = dst_agg.at[_EPM:_EPM + _EB].set(edge_index_boundary[1])

    ea_all = jnp.zeros((_EP, _KIN), _f32)
    ea_all = ea_all.at[:_E].set(edge_attr)
    ea_all = ea_all.at[_EPM:_EPM + _EB].set(edge_attr_boundary)

    k1, k2 = params["k1"], params["k2"]
    w1s = jnp.stack([k1[0][0], k2[0][0]])
    b1s = jnp.stack([k1[0][1], k2[0][1]])[:, None, :]
    w2s = jnp.stack([k1[1][0], k2[1][0]])
    b2s = jnp.stack([k1[1][1], k2[1][1]])[:, None, :]
    w3s = jnp.stack([k1[2][0], k2[2][0]])
    b3s = jnp.stack([k1[2][1].reshape(_W, _W),
                     k2[2][1].reshape(_W, _W)])

    g = _blockdiag4(params["fc1_w"])
    b1t = jnp.tile(params["fc1_b"], 4)[None, :]
    rblk = _blockdiag4(params["root1"] + params["root2"])
    bs128 = jnp.tile(params["bias1"] + params["bias2"], 4)[None, :]
    fblk = _blockdiag4(params["fc2_w"])
    fb = jnp.broadcast_to(params["fc2_b"].reshape(1, 1), (1, 4))

    rep = jnp.zeros((_W, _W * _W), _f32)
    for i in range(_W):
        rep = rep.at[i, i * _W:(i + 1) * _W].set(1.0)

    zeros_cnt = jnp.zeros((_NCNT, 16), _f32)
    zeros_agg = jnp.zeros((_NAGG, _W), _f32)
    ones_all = jnp.ones((_EP, 16), _f32)

    wbig = _edge_weights(ea_all, w1s, b1s, w2s, b2s, w3s)
    cnt = _scatter_rows(ones_all, dst_cnt, zeros_cnt, _NCNT)
    invtab = _inv_table(cnt)
    einv = _gather_rows(invtab, dst_cnt)
    x4 = x.reshape(_N // 4, 4)
    sk4 = _skip4(x4, g, b1t)

    def step_body(_, h4):
        h_rows = h4.reshape(_N, _W)
        xg = _gather_rows(h_rows, src_all)
        msg = _messages(xg, wbig, b3s, einv, rep)
        p = _scatter_rows(msg, dst_agg, zeros_agg, _NAGG)
        p0 = p[0, :_N].reshape(_N // 4, 128)
        p1 = p[1, :_N].reshape(_N // 4, 128)
        return _combine(p0, p1, h4, sk4, rblk, bs128)

    h4 = lax.fori_loop(0, _DEPTH, step_body, sk4)
    out = _final_proj(h4, fblk, fb)
    return out.reshape(_N, 1)


# speedup vs baseline: 1.3515x; 1.3515x over previous
import functools

import jax
import jax.numpy as jnp
from jax import lax
from jax.experimental import pallas as pl
from jax.experimental.pallas import tpu as pltpu
from jax.experimental.pallas import tpu_sc as plsc

_N = 10000
_E = 160000
_EB = 20000
_KIN = 6
_W = 32
_DEPTH = 4

_TILE = 512
_NTM = (_E + _TILE - 1) // _TILE
_NTB = (_EB + _TILE - 1) // _TILE
_NT = _NTM + _NTB
_EPM = _NTM * _TILE
_EP = _NT * _TILE

_NC = 2
_NS = 16
_NWK = _NC * _NS
_PER_W = _EP // _NWK
_CHUNK = _PER_W // 2
_NCHUNK = _PER_W // _CHUNK

_NCNT = 2 * _N + 16
_NAGG = _N + 16

_f32 = jnp.float32


def _sc_gather_body(h_hbm, src_hbm, out_hbm, idx_v, rows_v, sem):
    c = lax.axis_index("c")
    s = lax.axis_index("s")
    wid = s * _NC + c
    for k in range(_NCHUNK):
        base = wid * _PER_W + k * _CHUNK
        pltpu.sync_copy(src_hbm.at[pl.ds(base, _CHUNK)], idx_v)
        pltpu.async_copy(h_hbm.at[idx_v], rows_v, sem).wait()
        pltpu.sync_copy(rows_v, out_hbm.at[pl.ds(base, _CHUNK)])


def _sc_scatter_body(val_hbm, dst_hbm, z_hbm, out_hbm, val_v, dst_v, acc,
                     *, zrows):
    c = lax.axis_index("c")
    s = lax.axis_index("s")
    wid = s * _NC + c
    zb = s * zrows
    pltpu.sync_copy(z_hbm.at[pl.ds(zb, zrows)], acc.at[pl.ds(zb, zrows)])
    plsc.subcore_barrier()
    for k in range(_NCHUNK):
        base = wid * _PER_W + k * _CHUNK
        pltpu.sync_copy(dst_hbm.at[pl.ds(base, _CHUNK)], dst_v)
        pltpu.sync_copy(val_hbm.at[pl.ds(base, _CHUNK)], val_v)
        pltpu.sync_copy(val_v, acc.at[dst_v], add=True)
    plsc.subcore_barrier()
    pltpu.sync_copy(acc.at[pl.ds(zb, zrows)], out_hbm.at[c, pl.ds(zb, zrows)])


def _sc_mesh():
    return plsc.VectorSubcoreMesh(
        core_axis_name="c", subcore_axis_name="s",
        num_cores=_NC, num_subcores=_NS)


def _gather_rows(table, src_all):
    width = table.shape[1]
    call = pl.kernel(
        _sc_gather_body,
        out_type=jax.ShapeDtypeStruct((_EP, width), _f32),
        mesh=_sc_mesh(),
        compiler_params=pltpu.CompilerParams(use_tc_tiling_on_sc=False),
        scratch_types=[
            pltpu.VMEM((_CHUNK,), jnp.int32),
            pltpu.VMEM((_CHUNK, width), _f32),
            pltpu.SemaphoreType.DMA,
        ],
    )
    return call(table, src_all)


def _scatter_rows(vals, dst, zeros_acc, nacc):
    width = vals.shape[1]
    zrows = nacc // _NS
    body = functools.partial(_sc_scatter_body, zrows=zrows)
    call = pl.kernel(
        body,
        out_type=jax.ShapeDtypeStruct((_NC, nacc, width), _f32),
        mesh=_sc_mesh(),
        compiler_params=pltpu.CompilerParams(use_tc_tiling_on_sc=False),
        scratch_types=[
            pltpu.VMEM((_CHUNK, width), _f32),
            pltpu.VMEM((_CHUNK,), jnp.int32),
            pltpu.VMEM_SHARED((nacc, width), _f32),
        ],
    )
    return call(vals, dst, zeros_acc)


def _wbig_body(ea_ref, w1_ref, b1_ref, w2_ref, b2_ref, w3_ref, o_ref):
    t = jnp.dot(ea_ref[...], w1_ref[0], preferred_element_type=_f32) + b1_ref[0]
    t = jnp.maximum(t, 0.0)
    t = jnp.dot(t, w2_ref[0], preferred_element_type=_f32) + b2_ref[0]
    t = jnp.maximum(t, 0.0)
    o_ref[...] = jnp.dot(t, w3_ref[0],
                         preferred_element_type=_f32).astype(jnp.bfloat16)


def _sel(i):
    return jnp.where(i < _NTM, 0, 1)


def _edge_weights(ea_all, w1s, b1s, w2s, b2s, w3s):
    return pl.pallas_call(
        _wbig_body,
        grid=(_NT,),
        in_specs=[
            pl.BlockSpec((_TILE, _KIN), lambda i: (i, 0)),
            pl.BlockSpec((1, _KIN, 64), lambda i: (_sel(i), 0, 0)),
            pl.BlockSpec((1, 1, 64), lambda i: (_sel(i), 0, 0)),
            pl.BlockSpec((1, 64, 64), lambda i: (_sel(i), 0, 0)),
            pl.BlockSpec((1, 1, 64), lambda i: (_sel(i), 0, 0)),
            pl.BlockSpec((1, 64, _W * _W), lambda i: (_sel(i), 0, 0)),
        ],
        out_specs=pl.BlockSpec((_TILE, _W * _W), lambda i: (i, 0)),
        out_shape=jax.ShapeDtypeStruct((_EP, _W * _W), jnp.bfloat16),
    )(ea_all, w1s, b1s, w2s, b2s, w3s)


def _msg_body(xg_ref, w_ref, b3_ref, einv_ref, rep_ref, o_ref):
    xg = xg_ref[...]
    xrep = jnp.dot(xg, rep_ref[...], preferred_element_type=_f32)
    w = w_ref[...].astype(_f32)
    acc = xrep[:, 0:128] * w[:, 0:128]
    for q in range(1, 8):
        acc = acc + xrep[:, q * 128:(q + 1) * 128] * w[:, q * 128:(q + 1) * 128]
    msg = (acc[:, 0:32] + acc[:, 32:64] + acc[:, 64:96] + acc[:, 96:128]
           + jnp.dot(xg, b3_ref[0], preferred_element_type=_f32))
    o_ref[...] = msg * einv_ref[:, 0:1]


def _messages(xg, wbig, b3s, einv, rep):
    return pl.pallas_call(
        _msg_body,
        grid=(_NT,),
        in_specs=[
            pl.BlockSpec((_TILE, _W), lambda i: (i, 0)),
            pl.BlockSpec((_TILE, _W * _W), lambda i: (i, 0)),
            pl.BlockSpec((1, _W, _W), lambda i: (_sel(i), 0, 0)),
            pl.BlockSpec((_TILE, 16), lambda i: (i, 0)),
            pl.BlockSpec((_W, _W * _W), lambda i: (0, 0)),
        ],
        out_specs=pl.BlockSpec((_TILE, _W), lambda i: (i, 0)),
        out_shape=jax.ShapeDtypeStruct((_EP, _W), _f32),
    )(xg, wbig, b3s, einv, rep)


def _skip_body(x4_ref, g_ref, b_ref, o_ref):
    o_ref[...] = (jnp.dot(x4_ref[...], g_ref[...], preferred_element_type=_f32)
                  + b_ref[...])


def _skip4(x4, g, b128):
    return pl.pallas_call(
        _skip_body,
        out_shape=jax.ShapeDtypeStruct((_N // 4, 128), _f32),
    )(x4, g, b128)


def _combine_body(p0, p1, h4, sk4, r_ref, bs_ref, o_ref):
    y = (p0[...] + p1[...]
         + jnp.dot(h4[...], r_ref[...], preferred_element_type=_f32)
         + bs_ref[...])
    o_ref[...] = jnp.maximum(y, 0.0) + sk4[...]


def _combine(p0, p1, h4, sk4, rblk, bs128):
    return pl.pallas_call(
        _combine_body,
        out_shape=jax.ShapeDtypeStruct((_N // 4, 128), _f32),
    )(p0, p1, h4, sk4, rblk, bs128)


def _inv_body(c0, c1, o_ref):
    o_ref[...] = 1.0 / jnp.maximum(c0[...] + c1[...], 1.0)


def _inv_table(cnt):
    rows = _NCNT * 16 // 128
    c0 = cnt[0].reshape(rows, 128)
    c1 = cnt[1].reshape(rows, 128)
    inv = pl.pallas_call(
        _inv_body,
        out_shape=jax.ShapeDtypeStruct((rows, 128), _f32),
    )(c0, c1)
    return inv.reshape(_NCNT, 16)


def _final_body(h4, f_ref, fb_ref, o_ref):
    o_ref[...] = (jnp.dot(h4[...], f_ref[...], preferred_element_type=_f32)
                  + fb_ref[...])


def _final_proj(h4, fblk, fb):
    return pl.pallas_call(
        _final_body,
        out_shape=jax.ShapeDtypeStruct((_N // 4, 4), _f32),
    )(h4, fblk, fb)


def _blockdiag4(m):
    a, b = m.shape
    z = jnp.zeros((4, a, 4, b), m.dtype)
    for j in range(4):
        z = z.at[j, :, j, :].set(m)
    return z.reshape(4 * a, 4 * b)


def kernel(x, edge_index, edge_attr, edge_index_boundary, edge_attr_boundary,
           params):
    src_all = jnp.zeros((_EP,), jnp.int32)
    src_all = src_all.at[:_E].set(edge_index[0])
    src_all = src_all.at[_EPM:_EPM + _EB].set(edge_index_boundary[0])

    dst_cnt = jnp.full((_EP,), 2 * _N, jnp.int32)
    dst_cnt = dst_cnt.at[:_E].set(edge_index[1])
    dst_cnt = dst_cnt.at[_EPM:_EPM + _EB].set(edge_index_boundary[1] + _N)
    dst_agg = jnp.full((_EP,), _N, jnp.int32)
    dst_agg = dst_agg.at[:_E].set(edge_index[1])
    dst_agg = dst_agg.at[_EPM:_EPM + _EB].set(edge_index_boundary[1])

    ea_all = jnp.zeros((_EP, _KIN), _f32)
    ea_all = ea_all.at[:_E].set(edge_attr)
    ea_all = ea_all.at[_EPM:_EPM + _EB].set(edge_attr_boundary)

    k1, k2 = params["k1"], params["k2"]
    w1s = jnp.stack([k1[0][0], k2[0][0]])
    b1s = jnp.stack([k1[0][1], k2[0][1]])[:, None, :]
    w2s = jnp.stack([k1[1][0], k2[1][0]])
    b2s = jnp.stack([k1[1][1], k2[1][1]])[:, None, :]
    w3s = jnp.stack([k1[2][0], k2[2][0]])
    b3s = jnp.stack([k1[2][1].reshape(_W, _W),
                     k2[2][1].reshape(_W, _W)])

    g = _blockdiag4(params["fc1_w"])
    b1t = jnp.tile(params["fc1_b"], 4)[None, :]
    rblk = _blockdiag4(params["root1"] + params["root2"])
    bs128 = jnp.tile(params["bias1"] + params["bias2"], 4)[None, :]
    fblk = _blockdiag4(params["fc2_w"])
    fb = jnp.broadcast_to(params["fc2_b"].reshape(1, 1), (1, 4))

    rep = jnp.zeros((_W, _W * _W), _f32)
    for i in range(_W):
        rep = rep.at[i, i * _W:(i + 1) * _W].set(1.0)

    zeros_cnt = jnp.zeros((_NCNT, 16), _f32)
    zeros_agg = jnp.zeros((_NAGG, _W), _f32)
    ones_all = jnp.ones((_EP, 16), _f32)

    wbig = _edge_weights(ea_all, w1s, b1s, w2s, b2s, w3s)
    cnt = _scatter_rows(ones_all, dst_cnt, zeros_cnt, _NCNT)
    invtab = _inv_table(cnt)
    einv = _gather_rows(invtab, dst_cnt)
    x4 = x.reshape(_N // 4, 4)
    sk4 = _skip4(x4, g, b1t)

    def step_body(_, h4):
        h_rows = h4.reshape(_N, _W)
        xg = _gather_rows(h_rows, src_all)
        msg = _messages(xg, wbig, b3s, einv, rep)
        p = _scatter_rows(msg, dst_agg, zeros_agg, _NAGG)
        p0 = p[0, :_N].reshape(_N // 4, 128)
        p1 = p[1, :_N].reshape(_N // 4, 128)
        return _combine(p0, p1, h4, sk4, rblk, bs128)

    h4 = lax.fori_loop(0, _DEPTH, step_body, sk4)
    out = _final_proj(h4, fblk, fb)
    return out.reshape(_N, 1)
